# baseline (speedup 1.0000x reference)
import functools

import jax
import jax.numpy as jnp
from jax import lax
from jax.experimental import pallas as pl
from jax.experimental.pallas import tpu as pltpu
from jax.experimental.pallas import tpu_sc as plsc

VOCAB = 3
T_DIM = 4
TBL_ROWS = VOCAB ** T_DIM
NUM_CORES = 1
NUM_SUBCORES = 16
LANES = 16
NUM_WORKERS = NUM_CORES * NUM_SUBCORES


def _make_sc_gather(n_idx: int, tbl_len: int):
    chunk = n_idx // NUM_WORKERS
    groups = chunk // LANES

    mesh = plsc.VectorSubcoreMesh(
        core_axis_name="c", subcore_axis_name="s", num_cores=NUM_CORES
    )

    @functools.partial(
        pl.kernel,
        out_type=jax.ShapeDtypeStruct((n_idx * VOCAB,), jnp.float32),
        mesh=mesh,
        scratch_types=[
            pltpu.VMEM((chunk,), jnp.int32),
            pltpu.VMEM((tbl_len,), jnp.float32),
            pltpu.VMEM((chunk * VOCAB,), jnp.float32),
            pltpu.SemaphoreType.DMA,
            pltpu.SemaphoreType.DMA,
            pltpu.SemaphoreType.DMA,
        ],
        compiler_params=pltpu.CompilerParams(needs_layout_passes=False),
    )
    def sc_gather(idx_hbm, tbl_hbm, out_hbm, idx_v, tbl_v, out_v, s0, s1, s2):
        wid = lax.axis_index("s") * NUM_CORES + lax.axis_index("c")
        base = wid * chunk
        cp_idx = pltpu.async_copy(idx_hbm.at[pl.ds(base, chunk)], idx_v, s0)
        cp_tbl = pltpu.async_copy(tbl_hbm, tbl_v, s1)
        cp_idx.wait()
        cp_tbl.wait()

        @plsc.parallel_loop(0, groups, unroll=8)
        def body(g):
            jg = lax.shift_right_logical(g, 3) & 3
            scale = jnp.where(
                jg == 0, 1, jnp.where(jg == 1, 3, jnp.where(jg == 2, 9, 27))
            )
            iv = idx_v[pl.ds(g * LANES, LANES)]
            f = iv * scale
            for c in range(VOCAB):
                vals = plsc.load_gather(tbl_v, [f + c * TBL_ROWS])
                out_v[pl.ds(c * chunk + g * LANES, LANES)] = vals

        sems = (s0, s1, s2)
        cps = [
            pltpu.async_copy(
                out_v.at[pl.ds(c * chunk, chunk)],
                out_hbm.at[pl.ds(c * n_idx + base, chunk)],
                sems[c],
            )
            for c in range(VOCAB)
        ]
        for cp in cps:
            cp.wait()

    return sc_gather


def kernel(idx, outputs):
    b, t = idx.shape
    ihi = b // 128
    idx_nat = (
        idx.reshape(ihi, 128, t).transpose(0, 2, 1).reshape(-1).astype(jnp.int32)
    )
    tbl_cols = outputs.T.reshape(-1)
    out_flat = _make_sc_gather(b * t, tbl_cols.shape[0])(idx_nat, tbl_cols)
    return (
        out_flat.reshape(VOCAB, ihi, t, 128)
        .transpose(1, 3, 2, 0)
        .reshape(b, t, VOCAB)
    )
